# baseline (speedup 1.0000x reference)
import jax
import jax.numpy as jnp
from jax import lax
from jax.experimental import pallas as pl
from jax.experimental.pallas import tpu as pltpu
from jax.experimental.pallas import tpu_sc as plsc

N = 1024
E = 65536
OUT = 256
NC, NS = 2, 16
NW = NC * NS
L = 16
CHUNK = E // NW
NVEC = CHUNK // L
UNROLL = 4
R = 4


def _dot_nt(a, b):
    return lax.dot_general(a, b, (((1,), (1,)), ((), ())),
                           preferred_element_type=jnp.float32)


def _outer_rc(u, v):
    return lax.dot_general(u, v, (((0,), (1,)), ((), ())),
                           preferred_element_type=jnp.float32)


def _sc_agg_body(ei1, ei2, ei3, a_hbm, out,
                 a_v, s1v, d1v, s2v, d2v, s3v,
                 acc_s1, acc_c1, acc_s2, acc_c2, acc_c3, red, sem):
    wid = lax.axis_index("s") * NC + lax.axis_index("c")
    base = wid * CHUNK
    accs = (acc_s1, acc_c1, acc_s2, acc_c2, acc_c3)

    copies = [
        pltpu.async_copy(a_hbm.at[0], a_v, sem),
        pltpu.async_copy(ei1.at[0, pl.ds(base, CHUNK)], s1v, sem),
        pltpu.async_copy(ei1.at[1, pl.ds(base, CHUNK)], d1v, sem),
        pltpu.async_copy(ei2.at[0, pl.ds(base, CHUNK)], s2v, sem),
        pltpu.async_copy(ei2.at[1, pl.ds(base, CHUNK)], d2v, sem),
        pltpu.async_copy(ei3.at[0, pl.ds(base, CHUNK)], s3v, sem),
    ]

    zeros16 = jnp.zeros((L,), jnp.float32)
    ones16 = jnp.ones((L,), jnp.float32)
    lane = lax.iota(jnp.int32, L)
    row = lane & (R - 1)
    grp = lane // R
    masks = [grp == g for g in range(L // R)]

    @plsc.parallel_loop(0, N // L, unroll=8)
    def _(c):
        for acc in accs:
            for r in range(R):
                acc[r, pl.ds(c * L, L)] = zeros16

    for cp in copies:
        cp.wait()

    @plsc.parallel_loop(0, NVEC, unroll=8)
    def _(i):
        b = i * L
        sv1 = s1v[pl.ds(b, L)]
        dv1 = d1v[pl.ds(b, L)]
        av1 = plsc.load_gather(a_v, [sv1])
        sv2 = s2v[pl.ds(b, L)]
        dv2 = d2v[pl.ds(b, L)]
        av2 = plsc.load_gather(a_v, [sv2])
        sv3 = s3v[pl.ds(b, L)]
        for m in masks:
            plsc.addupdate_scatter(acc_s1, [row, dv1], av1, mask=m)
            plsc.addupdate_scatter(acc_c1, [row, dv1], ones16, mask=m)
            plsc.addupdate_scatter(acc_s2, [row, dv2], av2, mask=m)
            plsc.addupdate_scatter(acc_c2, [row, dv2], ones16, mask=m)
            plsc.addupdate_scatter(acc_c3, [row, sv3], ones16, mask=m)

    @plsc.parallel_loop(0, N // L, unroll=4)
    def _(c):
        for q, acc in enumerate(accs):
            s = acc[0, pl.ds(c * L, L)]
            for r in range(1, R):
                s = s + acc[r, pl.ds(c * L, L)]
            red[q, pl.ds(c * L, L)] = s

    pltpu.sync_copy(red, out.at[wid])


def _sc_agg(ei1, ei2, ei3, a_row):
    return pl.kernel(
        _sc_agg_body,
        out_type=jax.ShapeDtypeStruct((NW, 5, N), jnp.float32),
        mesh=plsc.VectorSubcoreMesh(core_axis_name="c", subcore_axis_name="s",
                                    num_cores=NC, num_subcores=NS),
        compiler_params=pltpu.CompilerParams(needs_layout_passes=False),
        scratch_types=[
            pltpu.VMEM((N,), jnp.float32),
            pltpu.VMEM((CHUNK,), jnp.int32),
            pltpu.VMEM((CHUNK,), jnp.int32),
            pltpu.VMEM((CHUNK,), jnp.int32),
            pltpu.VMEM((CHUNK,), jnp.int32),
            pltpu.VMEM((CHUNK,), jnp.int32),
            pltpu.VMEM((R, N), jnp.float32),
            pltpu.VMEM((R, N), jnp.float32),
            pltpu.VMEM((R, N), jnp.float32),
            pltpu.VMEM((R, N), jnp.float32),
            pltpu.VMEM((R, N), jnp.float32),
            pltpu.VMEM((5, N), jnp.float32),
            pltpu.SemaphoreType.DMA,
        ],
    )(ei1, ei2, ei3, a_row)


def _proj_body(art_ref, w1_ref, b1_ref, comm_ref, w2_ref, b2_ref,
               a_ref, cx_ref):
    a_ref[...] = _dot_nt(art_ref[...], w1_ref[...]) + b1_ref[...]
    cx_ref[...] = _dot_nt(comm_ref[...], w2_ref[...]) + b2_ref[...]


def _dense_body(p_ref, cx_ref, comm_ref,
                wl1_ref, bl1_ref, wr1_ref,
                wl2_ref, bl2_ref, wr2_ref,
                wl3_ref, bl3_ref, wr3_ref,
                w3_ref, b3_ref, out_ref):
    def rowsum(q):
        return jnp.sum(p_ref[:, q, :], axis=0, keepdims=True)

    s1, c1 = rowsum(0), rowsum(1)
    s2, c2 = rowsum(2), rowsum(3)
    cnt3 = rowsum(4)
    mean1 = s1 / jnp.maximum(c1, 1.0)
    mean2 = s2 / jnp.maximum(c2, 1.0)

    h1 = jnp.maximum(
        _outer_rc(mean1, wl1_ref[...]) + bl1_ref[...]
        + _outer_rc(cx_ref[...], wr1_ref[...]), 0.0)
    h2 = jnp.maximum(
        _outer_rc(mean2, wl2_ref[...]) + bl2_ref[...]
        + _dot_nt(h1, wr2_ref[...]), 0.0)
    mean3 = jnp.dot(cnt3, h2, preferred_element_type=jnp.float32) * (1.0 / E)
    h3 = jnp.maximum(
        _dot_nt(mean3, wl3_ref[...]) + bl3_ref[...]
        + _dot_nt(comm_ref[...], wr3_ref[...]), 0.0)
    out_ref[...] = _dot_nt(h3, w3_ref[...]) + b3_ref[...]


def kernel(article_x, community_x, ei_wb, ei_mb, ei_cc,
           W1, b1, W2, b2,
           Wl1, bl1, Wr1, Wl2, bl2, Wr2, Wl3, bl3, Wr3,
           W3, b3):
    f32 = jnp.float32

    a_row, cx_row = pl.pallas_call(
        _proj_body,
        out_shape=(jax.ShapeDtypeStruct((1, N), f32),
                   jax.ShapeDtypeStruct((1, N), f32)),
    )(article_x, W1, b1.reshape(1, N), community_x, W2, b2.reshape(1, N))

    parts = _sc_agg(ei_wb, ei_mb, ei_cc, a_row)

    out = pl.pallas_call(
        _dense_body,
        out_shape=jax.ShapeDtypeStruct((1, OUT), f32),
    )(parts, cx_row, community_x,
      Wl1, bl1.reshape(1, N), Wr1,
      Wl2, bl2.reshape(1, N), Wr2,
      Wl3, bl3.reshape(1, N), Wr3,
      W3, b3.reshape(1, OUT))
    return out


# speedup vs baseline: 1.0171x; 1.0171x over previous
import jax
import jax.numpy as jnp
from jax import lax
from jax.experimental import pallas as pl
from jax.experimental.pallas import tpu as pltpu
from jax.experimental.pallas import tpu_sc as plsc

N = 1024
E = 65536
OUT = 256
NC, NS = 2, 16
NW = NC * NS
L = 16
CHUNK = E // NW
NVEC = CHUNK // L
UNROLL = 4
R = 4


def _dot_nt(a, b):
    return lax.dot_general(a, b, (((1,), (1,)), ((), ())),
                           preferred_element_type=jnp.float32)


def _outer_rc(u, v):
    return lax.dot_general(u, v, (((0,), (1,)), ((), ())),
                           preferred_element_type=jnp.float32)


def _sc_agg_body(ei1, ei2, ei3, a_hbm, out,
                 a_v, s1v, d1v, s2v, d2v, s3v,
                 acc_s1, acc_c1, acc_s2, acc_c2, acc_c3, red, sem):
    wid = lax.axis_index("s") * NC + lax.axis_index("c")
    base = wid * CHUNK
    accs = (acc_s1, acc_c1, acc_s2, acc_c2, acc_c3)

    copies = [
        pltpu.async_copy(a_hbm.at[0], a_v, sem),
        pltpu.async_copy(ei1.at[0, pl.ds(base, CHUNK)], s1v, sem),
        pltpu.async_copy(ei1.at[1, pl.ds(base, CHUNK)], d1v, sem),
        pltpu.async_copy(ei2.at[0, pl.ds(base, CHUNK)], s2v, sem),
        pltpu.async_copy(ei2.at[1, pl.ds(base, CHUNK)], d2v, sem),
        pltpu.async_copy(ei3.at[0, pl.ds(base, CHUNK)], s3v, sem),
    ]

    zeros16 = jnp.zeros((L,), jnp.float32)
    ones16 = jnp.ones((L,), jnp.float32)
    lane = lax.iota(jnp.int32, L)
    row = lane & (R - 1)
    grp = lane // R
    masks = [grp == g for g in range(L // R)]

    @plsc.parallel_loop(0, N // L, unroll=4)
    def _(c):
        for acc in accs:
            for r in range(R):
                acc[r, pl.ds(c * L, L)] = zeros16

    for cp in copies:
        cp.wait()

    @plsc.parallel_loop(0, NVEC, unroll=UNROLL)
    def _(i):
        b = i * L
        sv1 = s1v[pl.ds(b, L)]
        dv1 = d1v[pl.ds(b, L)]
        av1 = plsc.load_gather(a_v, [sv1])
        sv2 = s2v[pl.ds(b, L)]
        dv2 = d2v[pl.ds(b, L)]
        av2 = plsc.load_gather(a_v, [sv2])
        sv3 = s3v[pl.ds(b, L)]
        for m in masks:
            plsc.addupdate_scatter(acc_s1, [row, dv1], av1, mask=m)
            plsc.addupdate_scatter(acc_c1, [row, dv1], ones16, mask=m)
            plsc.addupdate_scatter(acc_s2, [row, dv2], av2, mask=m)
            plsc.addupdate_scatter(acc_c2, [row, dv2], ones16, mask=m)
            plsc.addupdate_scatter(acc_c3, [row, sv3], ones16, mask=m)

    @plsc.parallel_loop(0, N // L, unroll=2)
    def _(c):
        for q, acc in enumerate(accs):
            s = acc[0, pl.ds(c * L, L)]
            for r in range(1, R):
                s = s + acc[r, pl.ds(c * L, L)]
            red[q, pl.ds(c * L, L)] = s

    pltpu.sync_copy(red, out.at[wid])


def _sc_agg(ei1, ei2, ei3, a_row):
    return pl.kernel(
        _sc_agg_body,
        out_type=jax.ShapeDtypeStruct((NW, 5, N), jnp.float32),
        mesh=plsc.VectorSubcoreMesh(core_axis_name="c", subcore_axis_name="s",
                                    num_cores=NC, num_subcores=NS),
        compiler_params=pltpu.CompilerParams(needs_layout_passes=False),
        scratch_types=[
            pltpu.VMEM((N,), jnp.float32),
            pltpu.VMEM((CHUNK,), jnp.int32),
            pltpu.VMEM((CHUNK,), jnp.int32),
            pltpu.VMEM((CHUNK,), jnp.int32),
            pltpu.VMEM((CHUNK,), jnp.int32),
            pltpu.VMEM((CHUNK,), jnp.int32),
            pltpu.VMEM((R, N), jnp.float32),
            pltpu.VMEM((R, N), jnp.float32),
            pltpu.VMEM((R, N), jnp.float32),
            pltpu.VMEM((R, N), jnp.float32),
            pltpu.VMEM((R, N), jnp.float32),
            pltpu.VMEM((5, N), jnp.float32),
            pltpu.SemaphoreType.DMA,
        ],
    )(ei1, ei2, ei3, a_row)


def _proj_body(art_ref, w1_ref, b1_ref, comm_ref, w2_ref, b2_ref,
               a_ref, cx_ref):
    a_ref[...] = _dot_nt(art_ref[...], w1_ref[...]) + b1_ref[...]
    cx_ref[...] = _dot_nt(comm_ref[...], w2_ref[...]) + b2_ref[...]


def _dense_body(p_ref, cx_ref, comm_ref,
                wl1_ref, bl1_ref, wr1_ref,
                wl2_ref, bl2_ref, wr2_ref,
                wl3_ref, bl3_ref, wr3_ref,
                w3_ref, b3_ref, out_ref):
    def rowsum(q):
        return jnp.sum(p_ref[:, q, :], axis=0, keepdims=True)

    s1, c1 = rowsum(0), rowsum(1)
    s2, c2 = rowsum(2), rowsum(3)
    cnt3 = rowsum(4)
    mean1 = s1 / jnp.maximum(c1, 1.0)
    mean2 = s2 / jnp.maximum(c2, 1.0)

    h1 = jnp.maximum(
        _outer_rc(mean1, wl1_ref[...]) + bl1_ref[...]
        + _outer_rc(cx_ref[...], wr1_ref[...]), 0.0)
    h2 = jnp.maximum(
        _outer_rc(mean2, wl2_ref[...]) + bl2_ref[...]
        + _dot_nt(h1, wr2_ref[...]), 0.0)
    mean3 = jnp.dot(cnt3, h2, preferred_element_type=jnp.float32) * (1.0 / E)
    h3 = jnp.maximum(
        _dot_nt(mean3, wl3_ref[...]) + bl3_ref[...]
        + _dot_nt(comm_ref[...], wr3_ref[...]), 0.0)
    out_ref[...] = _dot_nt(h3, w3_ref[...]) + b3_ref[...]


def kernel(article_x, community_x, ei_wb, ei_mb, ei_cc,
           W1, b1, W2, b2,
           Wl1, bl1, Wr1, Wl2, bl2, Wr2, Wl3, bl3, Wr3,
           W3, b3):
    f32 = jnp.float32

    a_row, cx_row = pl.pallas_call(
        _proj_body,
        out_shape=(jax.ShapeDtypeStruct((1, N), f32),
                   jax.ShapeDtypeStruct((1, N), f32)),
    )(article_x, W1, b1.reshape(1, N), community_x, W2, b2.reshape(1, N))

    parts = _sc_agg(ei_wb, ei_mb, ei_cc, a_row)

    out = pl.pallas_call(
        _dense_body,
        out_shape=jax.ShapeDtypeStruct((1, OUT), f32),
    )(parts, cx_row, community_x,
      Wl1, bl1.reshape(1, N), Wr1,
      Wl2, bl2.reshape(1, N), Wr2,
      Wl3, bl3.reshape(1, N), Wr3,
      W3, b3.reshape(1, OUT))
    return out


# speedup vs baseline: 1.0244x; 1.0072x over previous
import jax
import jax.numpy as jnp
from jax import lax
from jax.experimental import pallas as pl
from jax.experimental.pallas import tpu as pltpu
from jax.experimental.pallas import tpu_sc as plsc

N = 1024
E = 65536
OUT = 256
NC, NS = 2, 16
NW = NC * NS
L = 16
CHUNK = E // NW
NVEC = CHUNK // L
UNROLL = 2
R = 4


def _dot_nt(a, b):
    return lax.dot_general(a, b, (((1,), (1,)), ((), ())),
                           preferred_element_type=jnp.float32)


def _outer_rc(u, v):
    return lax.dot_general(u, v, (((0,), (1,)), ((), ())),
                           preferred_element_type=jnp.float32)


def _sc_agg_body(ei1, ei2, ei3, a_hbm, out,
                 a_v, s1v, d1v, s2v, d2v, s3v,
                 acc_s1, acc_c1, acc_s2, acc_c2, acc_c3, red, sem):
    wid = lax.axis_index("s") * NC + lax.axis_index("c")
    base = wid * CHUNK
    accs = (acc_s1, acc_c1, acc_s2, acc_c2, acc_c3)

    copies = [
        pltpu.async_copy(a_hbm.at[0], a_v, sem),
        pltpu.async_copy(ei1.at[0, pl.ds(base, CHUNK)], s1v, sem),
        pltpu.async_copy(ei1.at[1, pl.ds(base, CHUNK)], d1v, sem),
        pltpu.async_copy(ei2.at[0, pl.ds(base, CHUNK)], s2v, sem),
        pltpu.async_copy(ei2.at[1, pl.ds(base, CHUNK)], d2v, sem),
        pltpu.async_copy(ei3.at[0, pl.ds(base, CHUNK)], s3v, sem),
    ]

    zeros16 = jnp.zeros((L,), jnp.float32)
    ones16 = jnp.ones((L,), jnp.float32)
    lane = lax.iota(jnp.int32, L)
    row = lane & (R - 1)
    grp = lane // R
    masks = [grp == g for g in range(L // R)]

    @plsc.parallel_loop(0, N // L, unroll=2)
    def _(c):
        for acc in accs:
            for r in range(R):
                acc[r, pl.ds(c * L, L)] = zeros16

    for cp in copies:
        cp.wait()

    @plsc.parallel_loop(0, NVEC, unroll=UNROLL)
    def _(i):
        b = i * L
        sv1 = s1v[pl.ds(b, L)]
        dv1 = d1v[pl.ds(b, L)]
        av1 = plsc.load_gather(a_v, [sv1])
        sv2 = s2v[pl.ds(b, L)]
        dv2 = d2v[pl.ds(b, L)]
        av2 = plsc.load_gather(a_v, [sv2])
        sv3 = s3v[pl.ds(b, L)]
        for m in masks:
            plsc.addupdate_scatter(acc_s1, [row, dv1], av1, mask=m)
            plsc.addupdate_scatter(acc_c1, [row, dv1], ones16, mask=m)
            plsc.addupdate_scatter(acc_s2, [row, dv2], av2, mask=m)
            plsc.addupdate_scatter(acc_c2, [row, dv2], ones16, mask=m)
            plsc.addupdate_scatter(acc_c3, [row, sv3], ones16, mask=m)

    @plsc.parallel_loop(0, N // L, unroll=2)
    def _(c):
        for q, acc in enumerate(accs):
            s = acc[0, pl.ds(c * L, L)]
            for r in range(1, R):
                s = s + acc[r, pl.ds(c * L, L)]
            red[q, pl.ds(c * L, L)] = s

    pltpu.sync_copy(red, out.at[wid])


def _sc_agg(ei1, ei2, ei3, a_row):
    return pl.kernel(
        _sc_agg_body,
        out_type=jax.ShapeDtypeStruct((NW, 5, N), jnp.float32),
        mesh=plsc.VectorSubcoreMesh(core_axis_name="c", subcore_axis_name="s",
                                    num_cores=NC, num_subcores=NS),
        compiler_params=pltpu.CompilerParams(needs_layout_passes=False),
        scratch_types=[
            pltpu.VMEM((N,), jnp.float32),
            pltpu.VMEM((CHUNK,), jnp.int32),
            pltpu.VMEM((CHUNK,), jnp.int32),
            pltpu.VMEM((CHUNK,), jnp.int32),
            pltpu.VMEM((CHUNK,), jnp.int32),
            pltpu.VMEM((CHUNK,), jnp.int32),
            pltpu.VMEM((R, N), jnp.float32),
            pltpu.VMEM((R, N), jnp.float32),
            pltpu.VMEM((R, N), jnp.float32),
            pltpu.VMEM((R, N), jnp.float32),
            pltpu.VMEM((R, N), jnp.float32),
            pltpu.VMEM((5, N), jnp.float32),
            pltpu.SemaphoreType.DMA,
        ],
    )(ei1, ei2, ei3, a_row)


def _proj_body(art_ref, w1_ref, b1_ref, comm_ref, w2_ref, b2_ref,
               a_ref, cx_ref):
    a_ref[...] = _dot_nt(art_ref[...], w1_ref[...]) + b1_ref[...]
    cx_ref[...] = _dot_nt(comm_ref[...], w2_ref[...]) + b2_ref[...]


def _dense_body(p_ref, cx_ref, comm_ref,
                wl1_ref, bl1_ref, wr1_ref,
                wl2_ref, bl2_ref, wr2_ref,
                wl3_ref, bl3_ref, wr3_ref,
                w3_ref, b3_ref, out_ref):
    def rowsum(q):
        return jnp.sum(p_ref[:, q, :], axis=0, keepdims=True)

    s1, c1 = rowsum(0), rowsum(1)
    s2, c2 = rowsum(2), rowsum(3)
    cnt3 = rowsum(4)
    mean1 = s1 / jnp.maximum(c1, 1.0)
    mean2 = s2 / jnp.maximum(c2, 1.0)

    h1 = jnp.maximum(
        _outer_rc(mean1, wl1_ref[...]) + bl1_ref[...]
        + _outer_rc(cx_ref[...], wr1_ref[...]), 0.0)
    h2 = jnp.maximum(
        _outer_rc(mean2, wl2_ref[...]) + bl2_ref[...]
        + _dot_nt(h1, wr2_ref[...]), 0.0)
    mean3 = jnp.dot(cnt3, h2, preferred_element_type=jnp.float32) * (1.0 / E)
    h3 = jnp.maximum(
        _dot_nt(mean3, wl3_ref[...]) + bl3_ref[...]
        + _dot_nt(comm_ref[...], wr3_ref[...]), 0.0)
    out_ref[...] = _dot_nt(h3, w3_ref[...]) + b3_ref[...]


def kernel(article_x, community_x, ei_wb, ei_mb, ei_cc,
           W1, b1, W2, b2,
           Wl1, bl1, Wr1, Wl2, bl2, Wr2, Wl3, bl3, Wr3,
           W3, b3):
    f32 = jnp.float32

    a_row, cx_row = pl.pallas_call(
        _proj_body,
        out_shape=(jax.ShapeDtypeStruct((1, N), f32),
                   jax.ShapeDtypeStruct((1, N), f32)),
    )(article_x, W1, b1.reshape(1, N), community_x, W2, b2.reshape(1, N))

    parts = _sc_agg(ei_wb, ei_mb, ei_cc, a_row)

    out = pl.pallas_call(
        _dense_body,
        out_shape=jax.ShapeDtypeStruct((1, OUT), f32),
    )(parts, cx_row, community_x,
      Wl1, bl1.reshape(1, N), Wr1,
      Wl2, bl2.reshape(1, N), Wr2,
      Wl3, bl3.reshape(1, N), Wr3,
      W3, b3.reshape(1, OUT))
    return out
